# baseline (speedup 1.0000x reference)
import functools
import math

import jax
import jax.numpy as jnp
from jax import lax
from jax.experimental import pallas as pl
from jax.experimental.pallas import tpu as pltpu
from jax.experimental.pallas import tpu_sc as plsc

_LANES = 16
_PADW = 128
_CH = 128


@functools.lru_cache(maxsize=None)
def _build(batch: int, hist: int, hist_p: int, vocab: int, d_model: int,
           scale: float):
    info = plsc.get_sparse_core_info()
    nc, ns = info.num_cores, info.num_subcores
    nw = nc * ns
    assert batch % nw == 0
    tr_per_worker = batch // nw
    n_idx = tr_per_worker * hist
    assert n_idx % _CH == 0
    n_chunks = n_idx // _CH
    nbuf = 4
    assert n_chunks % nbuf == 0 and n_chunks >= 2 * nbuf
    n_groups = n_chunks // nbuf
    d_vecs = d_model // _LANES
    tr_half = tr_per_worker // 2
    assert tr_half * hist % _CH == 0
    half_chunks = tr_half * hist // _CH
    restage_g = half_chunks - 3

    mesh = plsc.VectorSubcoreMesh(core_axis_name="c", subcore_axis_name="s")

    @functools.partial(
        pl.kernel,
        out_type=jax.ShapeDtypeStruct((batch * hist, d_model), jnp.float32),
        mesh=mesh,
        scratch_types=[
            pltpu.VMEM((tr_half, hist_p), jnp.int32),
            pltpu.VMEM((nbuf * _CH,), jnp.int32),
            pltpu.VMEM((nbuf, _CH, _PADW), jnp.float32),
            pltpu.VMEM((_CH, d_model), jnp.float32),
            pltpu.VMEM((_CH, d_model), jnp.float32),
            [pltpu.SemaphoreType.DMA] * nbuf,
            [pltpu.SemaphoreType.DMA] * 2,
        ],
        compiler_params=pltpu.CompilerParams(
            use_tc_tiling_on_sc=True, needs_layout_passes=False
        ),
    )
    def gather_scale(t128_hbm, tok_hbm, out_hbm, idx_v, idx1d_v, rows_v,
                     pk0_v, pk1_v, gsems, wsems):
        wid = lax.axis_index("s") * nc + lax.axis_index("c")
        base_tr = wid * tr_per_worker
        base_flat = wid * n_idx
        pltpu.sync_copy(tok_hbm.at[pl.ds(base_tr, tr_half)], idx_v)

        def repack(w, slot):
            in_half2 = jnp.where(w >= half_chunks, tr_half, 0)
            for i in range(_CH // _LANES):
                p = w * _CH + i * _LANES + lax.iota(jnp.int32, _LANES)
                r = p // hist - in_half2
                c = p - (p // hist) * hist
                v = plsc.load_gather(idx_v, [r, c])
                idx1d_v[pl.ds(slot * _CH + i * _LANES, _LANES)] = v

        def gather(b):
            return pltpu.make_async_copy(
                t128_hbm.at[idx1d_v.at[pl.ds(b * _CH, _CH)]],
                rows_v.at[b],
                gsems[b],
            )

        def write(g, pk, bp):
            off = pl.multiple_of(base_flat + g * _CH, _CH)
            return pltpu.make_async_copy(
                pk, out_hbm.at[pl.ds(off, _CH)], wsems[bp]
            )

        repack(0, 0)
        repack(1, 1)
        repack(2, 2)
        gather(0).start()
        gather(1).start()

        def group_body(g0, _):
            for b in range(nbuf):
                g = g0 * nbuf + b
                bn = (b + 2) % nbuf
                pk = pk0_v if b % 2 == 0 else pk1_v

                if b == restage_g % nbuf:

                    @pl.when(g == restage_g)
                    def _():
                        pltpu.sync_copy(
                            tok_hbm.at[pl.ds(base_tr + tr_half, tr_half)],
                            idx_v,
                        )

                @pl.when(g + 3 < n_chunks)
                def _():
                    repack(g + 3, (b + 3) % nbuf)

                @pl.when(g >= 2)
                def _():
                    write(g - 2, pk, b % 2).wait()

                @pl.when(g + 2 < n_chunks)
                def _():
                    gather(bn).start()

                gather(b).wait()

                @plsc.parallel_loop(0, _CH, unroll=8)
                def _(r):
                    for d in range(d_vecs):
                        sl = pl.ds(d * _LANES, _LANES)
                        pk[r, sl] = rows_v[b, r, sl] * scale

                write(g, pk, b % 2).start()
            return 0

        lax.fori_loop(0, n_groups, group_body, 0)
        write(n_chunks - 2, pk0_v if (n_chunks - 2) % 2 == 0 else pk1_v,
              (n_chunks - 2) % 2).wait()
        write(n_chunks - 1, pk0_v if (n_chunks - 1) % 2 == 0 else pk1_v,
              (n_chunks - 1) % 2).wait()

    return gather_scale


def kernel(token, lookup_table):
    batch, hist = token.shape
    vocab, d_model = lookup_table.shape
    scale = math.sqrt(d_model)
    hist_p = -(-hist // _PADW) * _PADW
    tok_p = jnp.pad(token.astype(jnp.int32), ((0, 0), (0, hist_p - hist)))
    t128 = jnp.zeros((vocab, _PADW), jnp.float32).at[:, :d_model].set(
        lookup_table)
    fn = _build(batch, hist, hist_p, vocab, d_model, scale)
    out = fn(t128, tok_p)
    return out.reshape(batch, hist, d_model)


# speedup vs baseline: 1.0161x; 1.0161x over previous
import functools
import math

import jax
import jax.numpy as jnp
from jax import lax
from jax.experimental import pallas as pl
from jax.experimental.pallas import tpu as pltpu
from jax.experimental.pallas import tpu_sc as plsc

_LANES = 16
_PADW = 128
_CH = 128


@functools.lru_cache(maxsize=None)
def _build_widen(vocab: int, d_model: int, blk: int):
    def body(t_ref, o_ref):
        o_ref[...] = jnp.concatenate(
            [t_ref[...], jnp.zeros((blk, _PADW - d_model), jnp.float32)],
            axis=1,
        )

    return pl.pallas_call(
        body,
        grid=(vocab // blk,),
        in_specs=[pl.BlockSpec((blk, d_model), lambda i: (i, 0))],
        out_specs=pl.BlockSpec((blk, _PADW), lambda i: (i, 0)),
        out_shape=jax.ShapeDtypeStruct((vocab, _PADW), jnp.float32),
    )


@functools.lru_cache(maxsize=None)
def _build(batch: int, hist: int, hist_p: int, vocab: int, d_model: int,
           scale: float):
    info = plsc.get_sparse_core_info()
    nc, ns = info.num_cores, info.num_subcores
    nw = nc * ns
    assert batch % nw == 0
    tr_per_worker = batch // nw
    n_idx = tr_per_worker * hist
    assert n_idx % _CH == 0
    n_chunks = n_idx // _CH
    nbuf = 4
    assert n_chunks % nbuf == 0 and n_chunks >= 2 * nbuf
    n_groups = n_chunks // nbuf
    d_vecs = d_model // _LANES
    tr_half = tr_per_worker // 2
    assert tr_half * hist % _CH == 0
    half_chunks = tr_half * hist // _CH
    restage_g = half_chunks - 3

    mesh = plsc.VectorSubcoreMesh(core_axis_name="c", subcore_axis_name="s")

    @functools.partial(
        pl.kernel,
        out_type=jax.ShapeDtypeStruct((batch * hist, d_model), jnp.float32),
        mesh=mesh,
        scratch_types=[
            pltpu.VMEM((tr_half, hist_p), jnp.int32),
            pltpu.VMEM((nbuf * _CH,), jnp.int32),
            pltpu.VMEM((nbuf, _CH, _PADW), jnp.float32),
            pltpu.VMEM((_CH, d_model), jnp.float32),
            pltpu.VMEM((_CH, d_model), jnp.float32),
            [pltpu.SemaphoreType.DMA] * nbuf,
            [pltpu.SemaphoreType.DMA] * 2,
        ],
        compiler_params=pltpu.CompilerParams(
            use_tc_tiling_on_sc=True, needs_layout_passes=False
        ),
    )
    def gather_scale(t128_hbm, tok_hbm, out_hbm, idx_v, idx1d_v, rows_v,
                     pk0_v, pk1_v, gsems, wsems):
        wid = lax.axis_index("s") * nc + lax.axis_index("c")
        base_tr = wid * tr_per_worker
        base_flat = wid * n_idx
        pltpu.sync_copy(tok_hbm.at[pl.ds(base_tr, tr_half)], idx_v)

        def repack(w, slot):
            in_half2 = jnp.where(w >= half_chunks, tr_half, 0)
            for i in range(_CH // _LANES):
                p = w * _CH + i * _LANES + lax.iota(jnp.int32, _LANES)
                r = p // hist - in_half2
                c = p - (p // hist) * hist
                v = plsc.load_gather(idx_v, [r, c])
                idx1d_v[pl.ds(slot * _CH + i * _LANES, _LANES)] = v

        def gather(b):
            return pltpu.make_async_copy(
                t128_hbm.at[idx1d_v.at[pl.ds(b * _CH, _CH)]],
                rows_v.at[b],
                gsems[b],
            )

        def write(g, pk, bp):
            off = pl.multiple_of(base_flat + g * _CH, _CH)
            return pltpu.make_async_copy(
                pk, out_hbm.at[pl.ds(off, _CH)], wsems[bp]
            )

        repack(0, 0)
        repack(1, 1)
        repack(2, 2)
        gather(0).start()
        gather(1).start()

        def group_body(g0, _):
            for b in range(nbuf):
                g = g0 * nbuf + b
                bn = (b + 2) % nbuf
                pk = pk0_v if b % 2 == 0 else pk1_v

                if b == restage_g % nbuf:

                    @pl.when(g == restage_g)
                    def _():
                        pltpu.sync_copy(
                            tok_hbm.at[pl.ds(base_tr + tr_half, tr_half)],
                            idx_v,
                        )

                @pl.when(g + 3 < n_chunks)
                def _():
                    repack(g + 3, (b + 3) % nbuf)

                @pl.when(g >= 2)
                def _():
                    write(g - 2, pk, b % 2).wait()

                @pl.when(g + 2 < n_chunks)
                def _():
                    gather(bn).start()

                gather(b).wait()

                @plsc.parallel_loop(0, _CH, unroll=8)
                def _(r):
                    for d in range(d_vecs):
                        sl = pl.ds(d * _LANES, _LANES)
                        pk[r, sl] = rows_v[b, r, sl] * scale

                write(g, pk, b % 2).start()
            return 0

        lax.fori_loop(0, n_groups, group_body, 0)
        write(n_chunks - 2, pk0_v if (n_chunks - 2) % 2 == 0 else pk1_v,
              (n_chunks - 2) % 2).wait()
        write(n_chunks - 1, pk0_v if (n_chunks - 1) % 2 == 0 else pk1_v,
              (n_chunks - 1) % 2).wait()

    return gather_scale


def kernel(token, lookup_table):
    batch, hist = token.shape
    vocab, d_model = lookup_table.shape
    scale = math.sqrt(d_model)
    hist_p = -(-hist // _PADW) * _PADW
    tok_p = jnp.pad(token.astype(jnp.int32), ((0, 0), (0, hist_p - hist)))
    blk = 2000
    t128 = _build_widen(vocab, d_model, blk)(lookup_table)
    fn = _build(batch, hist, hist_p, vocab, d_model, scale)
    out = fn(t128, tok_p)
    return out.reshape(batch, hist, d_model)


# speedup vs baseline: 1.0484x; 1.0318x over previous
import functools
import math

import jax
import jax.numpy as jnp
from jax import lax
from jax.experimental import pallas as pl
from jax.experimental.pallas import tpu as pltpu
from jax.experimental.pallas import tpu_sc as plsc

_LANES = 16
_PADW = 128
_CH = 256


def _mesh():
    return plsc.VectorSubcoreMesh(core_axis_name="c", subcore_axis_name="s")


def _wid():
    return lax.axis_index("s") * plsc.get_sparse_core_info().num_cores + \
        lax.axis_index("c")


@functools.lru_cache(maxsize=None)
def _build_flatten(batch: int, hist: int, hist_p: int):
    info = plsc.get_sparse_core_info()
    nw = info.num_cores * info.num_subcores
    tr_per_worker = batch // nw
    tr_half = tr_per_worker // 2
    n_half = tr_half * hist
    assert n_half % _LANES == 0

    @functools.partial(
        pl.kernel,
        out_type=jax.ShapeDtypeStruct((batch * hist,), jnp.int32),
        mesh=_mesh(),
        scratch_types=[
            pltpu.VMEM((tr_half, hist_p), jnp.int32),
            pltpu.VMEM((n_half,), jnp.int32),
        ],
        compiler_params=pltpu.CompilerParams(
            use_tc_tiling_on_sc=True, needs_layout_passes=False
        ),
    )
    def flatten(tok_hbm, idx_hbm, tok_v, flat_v):
        wid = _wid()
        base_tr = wid * tr_per_worker
        for half in range(2):
            pltpu.sync_copy(
                tok_hbm.at[pl.ds(base_tr + half * tr_half, tr_half)], tok_v
            )

            def stripe(i, _):
                p = i * _LANES + lax.iota(jnp.int32, _LANES)
                r = p // hist
                c = p - r * hist
                v = plsc.load_gather(tok_v, [r, c])
                flat_v[pl.ds(pl.multiple_of(i * _LANES, _LANES), _LANES)] = v
                return 0

            lax.fori_loop(0, n_half // _LANES, stripe, 0)
            pltpu.sync_copy(
                flat_v,
                idx_hbm.at[pl.ds(wid * tr_per_worker * hist + half * n_half,
                                 n_half)],
            )

    return flatten


@functools.lru_cache(maxsize=None)
def _build_gather(n_rows: int, vocab: int, d_model: int, scale: float):
    info = plsc.get_sparse_core_info()
    nw = info.num_cores * info.num_subcores
    rows_per_worker = n_rows // nw
    assert rows_per_worker % _CH == 0
    n_chunks = rows_per_worker // _CH
    nbuf = 4
    assert n_chunks % nbuf == 0
    n_groups = n_chunks // nbuf
    d_vecs = d_model // _LANES
    pk = _CH * d_model // _PADW

    @functools.partial(
        pl.kernel,
        out_type=jax.ShapeDtypeStruct((n_rows * d_model // _PADW, _PADW),
                                      jnp.float32),
        mesh=_mesh(),
        scratch_types=[
            pltpu.VMEM((rows_per_worker,), jnp.int32),
            pltpu.VMEM((nbuf, _CH, d_model), jnp.float32),
            pltpu.VMEM((2, pk, _PADW), jnp.float32),
            [pltpu.SemaphoreType.DMA] * nbuf,
            [pltpu.SemaphoreType.DMA] * 2,
        ],
        compiler_params=pltpu.CompilerParams(use_tc_tiling_on_sc=False),
    )
    def gather_scale(table_hbm, idx_hbm, out_hbm, idx_v, rows_v, packed_v,
                     gsems, wsems):
        wid = _wid()
        base = wid * rows_per_worker
        base_pk = wid * (rows_per_worker * d_model // _PADW)
        pltpu.sync_copy(idx_hbm.at[pl.ds(base, rows_per_worker)], idx_v)

        def gather(g, b):
            off = pl.multiple_of(g * _CH, _CH)
            return pltpu.make_async_copy(
                table_hbm.at[idx_v.at[pl.ds(off, _CH)]], rows_v.at[b],
                gsems[b],
            )

        def write(g, bp):
            off = pl.multiple_of(base_pk + g * pk, pk)
            return pltpu.make_async_copy(
                packed_v.at[bp], out_hbm.at[pl.ds(off, pk)], wsems[bp]
            )

        gather(0, 0).start()
        gather(1, 1).start()

        def group_body(g0, _):
            for b in range(nbuf):
                g = g0 * nbuf + b
                bp = b % 2

                @pl.when(g >= 2)
                def _():
                    write(g - 2, bp).wait()

                @pl.when(g + 2 < n_chunks)
                def _():
                    gather(g + 2, (b + 2) % nbuf).start()

                gather(g, b).wait()

                @plsc.parallel_loop(0, pk, unroll=8)
                def _(j):
                    for h in range(2):
                        for d in range(d_vecs):
                            src = rows_v[b, 2 * j + h, pl.ds(d * _LANES,
                                                             _LANES)]
                            packed_v[bp, j,
                                     pl.ds(h * d_model + d * _LANES,
                                           _LANES)] = src * scale

                write(g, bp).start()
            return 0

        lax.fori_loop(0, n_groups, group_body, 0)
        write(n_chunks - 2, (n_chunks - 2) % 2).wait()
        write(n_chunks - 1, (n_chunks - 1) % 2).wait()

    return gather_scale


@functools.lru_cache(maxsize=None)
def _build_unpack(batch: int, hist: int, d_model: int):
    info = plsc.get_sparse_core_info()
    nw = info.num_cores * info.num_subcores
    tr_per_worker = batch // nw
    assert tr_per_worker % 2 == 0
    n_steps = tr_per_worker // 2
    pk_step = hist
    fl_step = 2 * hist
    d_vecs = d_model // _LANES

    @functools.partial(
        pl.kernel,
        out_type=jax.ShapeDtypeStruct((batch, hist, d_model), jnp.float32),
        mesh=_mesh(),
        scratch_types=[
            pltpu.VMEM((2, pk_step, _PADW), jnp.float32),
            pltpu.VMEM((2, hist, d_model), jnp.float32),
            [pltpu.SemaphoreType.DMA] * 2,
            pltpu.SemaphoreType.DMA,
        ],
        compiler_params=pltpu.CompilerParams(
            use_tc_tiling_on_sc=True, needs_layout_passes=False
        ),
    )
    def unpack(pk_hbm, out_hbm, in_v, out_v, rsems, wsem):
        wid = _wid()
        base_pk = wid * tr_per_worker * hist * d_model // _PADW
        base_tr = wid * tr_per_worker

        def read(s, b):
            off = pl.multiple_of(base_pk + s * pk_step, 8)
            return pltpu.make_async_copy(
                pk_hbm.at[pl.ds(off, pk_step)], in_v.at[b], rsems[b]
            )

        def write(s):
            return pltpu.make_async_copy(
                out_v, out_hbm.at[pl.ds(base_tr + s * 2, 2)], wsem
            )

        read(0, 0).start()

        def pair_body(s0, _):
            for b in range(2):
                s = s0 * 2 + b

                @pl.when(s + 1 < n_steps)
                def _():
                    read(s + 1, (b + 1) % 2).start()

                read(s, b).wait()

                @pl.when(s >= 1)
                def _():
                    write(s - 1).wait()

                @plsc.parallel_loop(0, fl_step, unroll=8)
                def _(fl):
                    src_r = fl // 2
                    src_h = (fl - src_r * 2) * d_model
                    t = fl // hist
                    h = fl - t * hist
                    for d in range(d_vecs):
                        out_v[t, h, pl.ds(d * _LANES, _LANES)] = in_v[
                            b, src_r, pl.ds(src_h + d * _LANES, _LANES)
                        ]

                write(s).start()
            return 0

        lax.fori_loop(0, n_steps // 2, pair_body, 0)
        write(n_steps - 1).wait()

    return unpack


def kernel(token, lookup_table):
    batch, hist = token.shape
    vocab, d_model = lookup_table.shape
    scale = math.sqrt(d_model)
    hist_p = -(-hist // _PADW) * _PADW
    tok_p = jnp.pad(token.astype(jnp.int32), ((0, 0), (0, hist_p - hist)))
    idx = _build_flatten(batch, hist, hist_p)(tok_p)
    pk = _build_gather(batch * hist, vocab, d_model, scale)(lookup_table, idx)
    return _build_unpack(batch, hist, d_model)(pk)


# speedup vs baseline: 1.1438x; 1.0910x over previous
import functools
import math

import jax
import jax.numpy as jnp
from jax import lax
from jax.experimental import pallas as pl
from jax.experimental.pallas import tpu as pltpu
from jax.experimental.pallas import tpu_sc as plsc

_LANES = 16
_PADW = 128
_CH = 256


def _mesh():
    return plsc.VectorSubcoreMesh(core_axis_name="c", subcore_axis_name="s")


def _wid():
    return lax.axis_index("s") * plsc.get_sparse_core_info().num_cores + \
        lax.axis_index("c")


@functools.lru_cache(maxsize=None)
def _build_flatten(batch: int, hist: int, hist_p: int):
    info = plsc.get_sparse_core_info()
    nw = info.num_cores * info.num_subcores
    tr_per_worker = batch // nw
    tr_half = tr_per_worker // 2
    n_half = tr_half * hist
    assert n_half % _PADW == 0
    fr_half = n_half // _PADW

    @functools.partial(
        pl.kernel,
        out_type=jax.ShapeDtypeStruct((batch * hist // _PADW, _PADW),
                                      jnp.int32),
        mesh=_mesh(),
        scratch_types=[
            pltpu.VMEM((tr_half, hist_p), jnp.int32),
            pltpu.VMEM((2 * fr_half, _PADW), jnp.int32),
        ],
        compiler_params=pltpu.CompilerParams(
            use_tc_tiling_on_sc=True, needs_layout_passes=False
        ),
    )
    def flatten(tok_hbm, idx_hbm, tok_v, flat_v):
        wid = _wid()
        base_tr = wid * tr_per_worker
        for half in range(2):
            pltpu.sync_copy(
                tok_hbm.at[pl.ds(base_tr + half * tr_half, tr_half)], tok_v
            )

            def row_body(fr, _):
                for j in range(_PADW // _LANES):
                    p = fr * _PADW + j * _LANES + lax.iota(jnp.int32, _LANES)
                    r = p // hist - half * tr_half
                    c = p - (p // hist) * hist
                    v = plsc.load_gather(tok_v, [r, c])
                    flat_v[fr, pl.ds(j * _LANES, _LANES)] = v
                return 0

            lax.fori_loop(half * fr_half, (half + 1) * fr_half, row_body, 0)
        pltpu.sync_copy(
            flat_v, idx_hbm.at[pl.ds(wid * 2 * fr_half, 2 * fr_half)]
        )

    return flatten


@functools.lru_cache(maxsize=None)
def _build_gather(n_rows: int, vocab: int, d_model: int, scale: float):
    info = plsc.get_sparse_core_info()
    nw = info.num_cores * info.num_subcores
    rows_per_worker = n_rows // nw
    ch = _PADW
    assert rows_per_worker % ch == 0
    n_chunks = rows_per_worker // ch
    nbuf = 4
    assert n_chunks % nbuf == 0
    n_groups = n_chunks // nbuf
    d_vecs = d_model // _LANES
    pk = ch * d_model // _PADW

    @functools.partial(
        pl.kernel,
        out_type=jax.ShapeDtypeStruct((n_rows * d_model // _PADW, _PADW),
                                      jnp.float32),
        mesh=_mesh(),
        scratch_types=[
            pltpu.VMEM((rows_per_worker // ch, _PADW), jnp.int32),
            pltpu.VMEM((nbuf, ch, d_model), jnp.float32),
            pltpu.VMEM((2, pk, _PADW), jnp.float32),
            [pltpu.SemaphoreType.DMA] * nbuf,
            [pltpu.SemaphoreType.DMA] * 2,
        ],
        compiler_params=pltpu.CompilerParams(use_tc_tiling_on_sc=False),
    )
    def gather_scale(table_hbm, idx_hbm, out_hbm, idx_v, rows_v, packed_v,
                     gsems, wsems):
        wid = _wid()
        base_ir = wid * n_chunks
        base_pk = wid * (rows_per_worker * d_model // _PADW)
        pltpu.sync_copy(idx_hbm.at[pl.ds(base_ir, n_chunks)], idx_v)

        def gather(g, b):
            return pltpu.make_async_copy(
                table_hbm.at[idx_v.at[g]], rows_v.at[b],
                gsems[b],
            )

        def write(g, bp):
            off = pl.multiple_of(base_pk + g * pk, pk)
            return pltpu.make_async_copy(
                packed_v.at[bp], out_hbm.at[pl.ds(off, pk)], wsems[bp]
            )

        gather(0, 0).start()
        gather(1, 1).start()

        def group_body(g0, _):
            for b in range(nbuf):
                g = g0 * nbuf + b
                bp = b % 2

                @pl.when(g >= 2)
                def _():
                    write(g - 2, bp).wait()

                @pl.when(g + 2 < n_chunks)
                def _():
                    gather(g + 2, (b + 2) % nbuf).start()

                gather(g, b).wait()

                @plsc.parallel_loop(0, pk, unroll=8)
                def _(j):
                    for h in range(2):
                        for d in range(d_vecs):
                            src = rows_v[b, 2 * j + h, pl.ds(d * _LANES,
                                                             _LANES)]
                            packed_v[bp, j,
                                     pl.ds(h * d_model + d * _LANES,
                                           _LANES)] = src * scale

                write(g, bp).start()
            return 0

        lax.fori_loop(0, n_groups, group_body, 0)
        write(n_chunks - 2, (n_chunks - 2) % 2).wait()
        write(n_chunks - 1, (n_chunks - 1) % 2).wait()

    return gather_scale


@functools.lru_cache(maxsize=None)
def _build_unpack(batch: int, hist: int, d_model: int):
    info = plsc.get_sparse_core_info()
    nw = info.num_cores * info.num_subcores
    tr_per_worker = batch // nw
    assert tr_per_worker % 2 == 0
    n_steps = tr_per_worker // 2
    pk_step = hist
    fl_step = 2 * hist
    d_vecs = d_model // _LANES

    @functools.partial(
        pl.kernel,
        out_type=jax.ShapeDtypeStruct((batch * hist, d_model), jnp.float32),
        mesh=_mesh(),
        scratch_types=[
            pltpu.VMEM((2, pk_step, _PADW), jnp.float32),
            pltpu.VMEM((fl_step, d_model), jnp.float32),
            [pltpu.SemaphoreType.DMA] * 2,
            pltpu.SemaphoreType.DMA,
        ],
        compiler_params=pltpu.CompilerParams(
            use_tc_tiling_on_sc=True, needs_layout_passes=False
        ),
    )
    def unpack(pk_hbm, out_hbm, in_v, out_v, rsems, wsem):
        wid = _wid()
        base_pk = wid * tr_per_worker * hist * d_model // _PADW
        base_fl = wid * tr_per_worker * hist

        def read(s, b):
            off = pl.multiple_of(base_pk + s * pk_step, 8)
            return pltpu.make_async_copy(
                pk_hbm.at[pl.ds(off, pk_step)], in_v.at[b], rsems[b]
            )

        def write(s):
            off = pl.multiple_of(base_fl + s * fl_step, 8)
            return pltpu.make_async_copy(
                out_v, out_hbm.at[pl.ds(off, fl_step)], wsem
            )

        read(0, 0).start()

        def pair_body(s0, _):
            for b in range(2):
                s = s0 * 2 + b

                @pl.when(s + 1 < n_steps)
                def _():
                    read(s + 1, (b + 1) % 2).start()

                read(s, b).wait()

                @pl.when(s >= 1)
                def _():
                    write(s - 1).wait()

                @plsc.parallel_loop(0, fl_step, unroll=8)
                def _(fl):
                    src_r = fl // 2
                    src_h = (fl - src_r * 2) * d_model
                    for d in range(d_vecs):
                        out_v[fl, pl.ds(d * _LANES, _LANES)] = in_v[
                            b, src_r, pl.ds(src_h + d * _LANES, _LANES)
                        ]

                write(s).start()
            return 0

        lax.fori_loop(0, n_steps // 2, pair_body, 0)
        write(n_steps - 1).wait()

    return unpack


def kernel(token, lookup_table):
    batch, hist = token.shape
    vocab, d_model = lookup_table.shape
    scale = math.sqrt(d_model)
    hist_p = -(-hist // _PADW) * _PADW
    tok_p = jnp.pad(token.astype(jnp.int32), ((0, 0), (0, hist_p - hist)))
    idx = _build_flatten(batch, hist, hist_p)(tok_p)
    pk = _build_gather(batch * hist, vocab, d_model, scale)(lookup_table, idx)
    out = _build_unpack(batch, hist, d_model)(pk)
    return out.reshape(batch, hist, d_model)
